# baseline (speedup 1.0000x reference)
import jax
import jax.numpy as jnp
from jax import lax
from jax.experimental import pallas as pl
from jax.experimental.pallas import tpu as pltpu
from jax.experimental.pallas import tpu_sc as plsc

_D = 32
_W = 200
_NC = 2
_NS = 16
_NW = _NC * _NS
_CHUNK = 800
_G = 100
_GPC = _CHUNK // _G
_NB = 4


def _embed_body(table_hbm, idx_hbm, pe_hbm, out_hbm, *refs):
    idx = refs[0:_NB]
    rows = refs[_NB:2 * _NB]
    psem = refs[2 * _NB:3 * _NB]
    gsem = refs[3 * _NB:4 * _NB]
    wsem = refs[4 * _NB:5 * _NB]

    n_total = idx_hbm.shape[0] * idx_hbm.shape[1]
    per_w = n_total // _NW
    n_chunks = per_w // _CHUNK
    wid = lax.axis_index("s") * _NC + lax.axis_index("c")
    base_row = wid * (per_w // _G)

    def stage(j, b):
        pltpu.sync_copy(idx_hbm.at[pl.ds(base_row + j * _GPC, _GPC)], idx[b])
        pltpu.async_copy(pe_hbm, rows[b], psem[b])

    def fire(j, b):
        pltpu.make_async_copy(pe_hbm, rows[b], psem[b]).wait()
        for g in range(_GPC):
            pltpu.async_copy(
                table_hbm.at[idx[b].at[g]],
                rows[b].at[pl.ds(g * _G, _G)],
                gsem[b],
                add=True,
            )

    def retire(j, b):
        pltpu.make_async_copy(table_hbm.at[pl.ds(0, _CHUNK)], rows[b], gsem[b]).wait()
        pltpu.async_copy(
            rows[b], out_hbm.at[pl.ds(wid * per_w + j * _CHUNK, _CHUNK)], wsem[b])

    def drain_w(b):
        pltpu.make_async_copy(rows[b], out_hbm.at[pl.ds(0, _CHUNK)], wsem[b]).wait()

    stage(0, 0)
    stage(1, 1)

    @pl.loop(0, n_chunks // _NB)
    def _group(q):
        for s in range(_NB):
            j = _NB * q + s
            bp = (s + _NB - 1) % _NB
            bn = (s + 2) % _NB

            fire(j, s)

            @pl.when(j > 0)
            def _():
                retire(j - 1, bp)

            @pl.when(j < n_chunks - 2)
            def _():
                @pl.when(j >= 2)
                def _():
                    drain_w(bn)

                stage(j + 2, bn)

    retire(n_chunks - 1, (n_chunks - 1) % _NB)
    for b in range(_NB):
        drain_w(b)


def _make_sc_call(n_total):
    mesh = plsc.VectorSubcoreMesh(
        core_axis_name="c", subcore_axis_name="s",
        num_cores=_NC, num_subcores=_NS,
    )
    return pl.kernel(
        _embed_body,
        out_type=jax.ShapeDtypeStruct((n_total, _D), jnp.float32),
        mesh=mesh,
        scratch_types=(
            [pltpu.VMEM((_GPC, _G), jnp.int32) for _ in range(_NB)]
            + [pltpu.VMEM((_CHUNK, _D), jnp.float32) for _ in range(_NB)]
            + [pltpu.SemaphoreType.DMA for _ in range(3 * _NB)]
        ),
        compiler_params=pltpu.CompilerParams(use_tc_tiling_on_sc=False),
    )


def kernel(vector, table, pe):
    b, w = vector.shape
    n_total = b * w
    idx = vector.reshape(n_total // _G, _G).astype(jnp.int32)
    pe_tile = jnp.tile(pe, (_CHUNK // _W, 1))
    out = _make_sc_call(n_total)(table, idx, pe_tile)
    return out.reshape(b, w, _D)


# speedup vs baseline: 1.1737x; 1.1737x over previous
import jax
import jax.numpy as jnp
from jax import lax
from jax.experimental import pallas as pl
from jax.experimental.pallas import tpu as pltpu
from jax.experimental.pallas import tpu_sc as plsc

_D = 32
_W = 200
_NC = 2
_NS = 16
_NW = _NC * _NS
_CHUNK = 800
_G = 100
_GPC = _CHUNK // _G
_NB = 4


def _embed_body(table_hbm, idx_hbm, pe_hbm, out_hbm, *refs):
    idx = refs[0:_NB]
    rows = refs[_NB:2 * _NB]
    pe_sh = refs[2 * _NB]
    psem = refs[2 * _NB + 1:3 * _NB + 1]
    gsem = refs[3 * _NB + 1:4 * _NB + 1]
    wsem = refs[4 * _NB + 1:5 * _NB + 1]

    n_total = idx_hbm.shape[0] * idx_hbm.shape[1]
    per_w = n_total // _NW
    n_chunks = per_w // _CHUNK
    wid = lax.axis_index("s") * _NC + lax.axis_index("c")
    base_row = wid * (per_w // _G)

    @pl.when(lax.axis_index("s") == 0)
    def _():
        pltpu.sync_copy(pe_hbm, pe_sh)

    plsc.subcore_barrier()

    def stage(j, b):
        pltpu.sync_copy(idx_hbm.at[pl.ds(base_row + j * _GPC, _GPC)], idx[b])
        pltpu.async_copy(pe_sh, rows[b], psem[b])

    def fire(j, b):
        pltpu.make_async_copy(pe_sh, rows[b], psem[b]).wait()
        for g in range(_GPC):
            pltpu.async_copy(
                table_hbm.at[idx[b].at[g]],
                rows[b].at[pl.ds(g * _G, _G)],
                gsem[b],
                add=True,
            )

    def retire(j, b):
        pltpu.make_async_copy(table_hbm.at[pl.ds(0, _CHUNK)], rows[b], gsem[b]).wait()
        pltpu.async_copy(
            rows[b], out_hbm.at[pl.ds(wid * per_w + j * _CHUNK, _CHUNK)], wsem[b])

    def drain_w(b):
        pltpu.make_async_copy(rows[b], out_hbm.at[pl.ds(0, _CHUNK)], wsem[b]).wait()

    stage(0, 0)
    stage(1, 1)

    @pl.loop(0, n_chunks // _NB)
    def _group(q):
        for s in range(_NB):
            j = _NB * q + s
            bp = (s + _NB - 1) % _NB
            bn = (s + 2) % _NB

            fire(j, s)

            @pl.when(j > 0)
            def _():
                retire(j - 1, bp)

            @pl.when(j < n_chunks - 2)
            def _():
                @pl.when(j >= 2)
                def _():
                    drain_w(bn)

                stage(j + 2, bn)

    retire(n_chunks - 1, (n_chunks - 1) % _NB)
    for b in range(_NB):
        drain_w(b)


def _make_sc_call(n_total):
    mesh = plsc.VectorSubcoreMesh(
        core_axis_name="c", subcore_axis_name="s",
        num_cores=_NC, num_subcores=_NS,
    )
    return pl.kernel(
        _embed_body,
        out_type=jax.ShapeDtypeStruct((n_total, _D), jnp.float32),
        mesh=mesh,
        scratch_types=(
            [pltpu.VMEM((_GPC, _G), jnp.int32) for _ in range(_NB)]
            + [pltpu.VMEM((_CHUNK, _D), jnp.float32) for _ in range(_NB)]
            + [pltpu.VMEM_SHARED((_CHUNK, _D), jnp.float32)]
            + [pltpu.SemaphoreType.DMA for _ in range(3 * _NB)]
        ),
        compiler_params=pltpu.CompilerParams(use_tc_tiling_on_sc=False),
    )


def kernel(vector, table, pe):
    b, w = vector.shape
    n_total = b * w
    idx = vector.reshape(n_total // _G, _G).astype(jnp.int32)
    pe_tile = jnp.tile(pe, (_CHUNK // _W, 1))
    out = _make_sc_call(n_total)(table, idx, pe_tile)
    return out.reshape(b, w, _D)
